# Initial kernel scaffold; baseline (speedup 1.0000x reference)
import functools

import jax
import jax.numpy as jnp
from jax import lax
from jax.experimental import pallas as pl
from jax.experimental.pallas import tpu as pltpu
from jax.experimental.pallas import tpu_sc as plsc

N = 10000
E = 320000
D = 128
DOUT = 12

NC = 2
NS = 16
NW = NC * NS
K = 128
CHUNKS = -(-E // (NW * K))
EPAD = CHUNKS * NW * K
NPAD = ((N + NS) // NS + 1) * NS
RPT = NPAD // NS

_mesh = plsc.VectorSubcoreMesh(core_axis_name="c", subcore_axis_name="s")

_ZERO16 = jnp.zeros((16,), jnp.float32)
_ONE16 = jnp.ones((16,), jnp.float32)


def _fill_2d(ref, rows, val16):
    def body(i, _):
        r = i // (D // 16)
        col = (i % (D // 16)) * 16
        ref[r, pl.ds(col, 16)] = val16
        return 0
    lax.fori_loop(0, rows * (D // 16), body, 0)


def _fill_1d(ref, n, val16):
    def body(i, _):
        ref[pl.ds(i * 16, 16)] = val16
        return 0
    lax.fori_loop(0, n // 16, body, 0)


@functools.partial(
    pl.kernel,
    out_type=jax.ShapeDtypeStruct((NC, NPAD), jnp.float32),
    mesh=_mesh,
    scratch_types=[
        pltpu.VMEM((K,), jnp.int32),
        pltpu.VMEM((K,), jnp.float32),
        pltpu.VMEM((RPT,), jnp.float32),
        pltpu.VMEM_SHARED((NPAD,), jnp.float32),
        pltpu.SemaphoreType.DMA,
    ],
)
def _sc_degree(dst_hbm, out_hbm, dst_v, ones_v, zrow_v, acc, sem):
    c = lax.axis_index("c")
    s = lax.axis_index("s")
    wid = s * NC + c

    _fill_1d(ones_v, K, _ONE16)
    _fill_1d(zrow_v, RPT, _ZERO16)
    pltpu.sync_copy(zrow_v, acc.at[pl.ds(s * RPT, RPT)])
    plsc.subcore_barrier()

    base = wid * CHUNKS * K

    def body(j, _):
        pltpu.sync_copy(dst_hbm.at[pl.ds(base + j * K, K)], dst_v)
        pltpu.sync_copy(ones_v, acc.at[dst_v], add=True)
        return 0

    lax.fori_loop(0, CHUNKS, body, 0)
    plsc.subcore_barrier()
    pltpu.sync_copy(acc.at[pl.ds(s * RPT, RPT)], out_hbm.at[c, pl.ds(s * RPT, RPT)])


@functools.partial(
    pl.kernel,
    out_type=jax.ShapeDtypeStruct((NC, NPAD, D), jnp.float32),
    mesh=_mesh,
    scratch_types=[
        pltpu.VMEM((K,), jnp.int32),
        pltpu.VMEM((K,), jnp.int32),
        pltpu.VMEM((K, D), jnp.float32),
        pltpu.VMEM_SHARED((NPAD, D), jnp.float32),
        pltpu.SemaphoreType.DMA,
    ],
)
def _sc_segsum(y_hbm, src_hbm, dst_hbm, out_hbm, src_v, dst_v, rows_v, acc, sem):
    c = lax.axis_index("c")
    s = lax.axis_index("s")
    wid = s * NC + c

    _fill_2d(rows_v, K, _ZERO16)
    rbase = s * RPT
    for j in range(RPT // K):
        pltpu.sync_copy(rows_v, acc.at[pl.ds(rbase + j * K, K)])
    rem = RPT % K
    if rem:
        pltpu.sync_copy(rows_v.at[pl.ds(0, rem)],
                        acc.at[pl.ds(rbase + (RPT // K) * K, rem)])
    plsc.subcore_barrier()

    base = wid * CHUNKS * K

    def body(j, _):
        off = base + j * K
        pltpu.sync_copy(src_hbm.at[pl.ds(off, K)], src_v)
        pltpu.sync_copy(dst_hbm.at[pl.ds(off, K)], dst_v)
        pltpu.async_copy(y_hbm.at[src_v], rows_v, sem).wait()
        pltpu.sync_copy(rows_v, acc.at[dst_v], add=True)
        return 0

    lax.fori_loop(0, CHUNKS, body, 0)
    plsc.subcore_barrier()
    pltpu.sync_copy(acc.at[pl.ds(rbase, RPT)], out_hbm.at[c, pl.ds(rbase, RPT)])


_R = 1000


def _tc_dense1_body(x_ref, w_ref, c0_ref, c1_ref, y_ref, dinv_ref):
    deg = c0_ref[...] + c1_ref[...] + 1.0
    dinv = lax.rsqrt(deg)
    dinv_ref[...] = dinv
    y_ref[...] = jnp.dot(x_ref[...], w_ref[...],
                         preferred_element_type=jnp.float32) * dinv


def _tc_dense_mid_body(z0_ref, z1_ref, y_ref, dinv_ref, b_ref, w_ref, out_ref):
    dinv = dinv_ref[...]
    h = dinv * (z0_ref[...] + z1_ref[...] + y_ref[...]) + b_ref[...]
    h = jnp.maximum(h, 0.0)
    out_ref[...] = jnp.dot(h, w_ref[...],
                           preferred_element_type=jnp.float32) * dinv


def _tc_dense_out_body(z0_ref, z1_ref, y_ref, dinv_ref, b_ref, m_ref,
                       wr_ref, br_ref, out_ref):
    h = dinv_ref[...] * (z0_ref[...] + z1_ref[...] + y_ref[...]) + b_ref[...]
    h = jnp.maximum(h, 0.0) * m_ref[...]
    out_ref[...] = jnp.dot(h, wr_ref[...],
                           preferred_element_type=jnp.float32) + br_ref[...]


def _rows_spec(width):
    return pl.BlockSpec((_R, width), lambda i: (i, 0))


def _full_spec(shape):
    return pl.BlockSpec(shape, lambda i: (0,) * len(shape))


def kernel(x, edge_index, target_mask, W1, b1, W2, b2, Wr, br):
    src = edge_index[0]
    dst = edge_index[1]
    pad = EPAD - E
    src_p = jnp.concatenate([src, jnp.zeros((pad,), jnp.int32)])
    dst_p = jnp.concatenate([dst, jnp.full((pad,), N, jnp.int32)])

    cnt = _sc_degree(dst_p)
    c0 = cnt[0, :N, None]
    c1 = cnt[1, :N, None]

    grid = N // _R
    y1, dinv = pl.pallas_call(
        _tc_dense1_body,
        grid=(grid,),
        in_specs=[_rows_spec(D), _full_spec((D, D)), _rows_spec(1), _rows_spec(1)],
        out_specs=[_rows_spec(D), _rows_spec(1)],
        out_shape=[jax.ShapeDtypeStruct((N, D), jnp.float32),
                   jax.ShapeDtypeStruct((N, 1), jnp.float32)],
    )(x, W1, c0, c1)

    z = _sc_segsum(y1, src_p, dst_p)

    y2 = pl.pallas_call(
        _tc_dense_mid_body,
        grid=(grid,),
        in_specs=[_rows_spec(D), _rows_spec(D), _rows_spec(D), _rows_spec(1),
                  _full_spec((1, D)), _full_spec((D, D))],
        out_specs=_rows_spec(D),
        out_shape=jax.ShapeDtypeStruct((N, D), jnp.float32),
    )(z[0, :N], z[1, :N], y1, dinv, b1[None, :], W2)

    z2 = _sc_segsum(y2, src_p, dst_p)

    mask_f = target_mask.astype(jnp.float32)[:, None]
    pred = pl.pallas_call(
        _tc_dense_out_body,
        grid=(grid,),
        in_specs=[_rows_spec(D), _rows_spec(D), _rows_spec(D), _rows_spec(1),
                  _full_spec((1, D)), _rows_spec(1),
                  _full_spec((D, DOUT)), _full_spec((1, DOUT))],
        out_specs=_rows_spec(DOUT),
        out_shape=jax.ShapeDtypeStruct((N, DOUT), jnp.float32),
    )(z2[0, :N], z2[1, :N], y2, dinv, b2[None, :], mask_f, Wr, br[None, :])

    return pred


# speedup vs baseline: 11.4325x; 11.4325x over previous
import functools

import jax
import jax.numpy as jnp
from jax import lax
from jax.experimental import pallas as pl
from jax.experimental.pallas import tpu as pltpu
from jax.experimental.pallas import tpu_sc as plsc

N = 10000
E = 320000
D = 128
DOUT = 12

NC = 2
NS = 16
NW = NC * NS
K = 128
CHUNKS = -(-E // (NW * K))
EPAD = CHUNKS * NW * K
NPAD = (N // 128 + 1) * 128
RPT = NPAD // NS

_mesh = plsc.VectorSubcoreMesh(core_axis_name="c", subcore_axis_name="s")

def _fill_2d(ref, rows, val):
    val16 = jnp.full((16,), val, jnp.float32)
    def body(i, _):
        r = i // (D // 16)
        col = (i % (D // 16)) * 16
        ref[r, pl.ds(col, 16)] = val16
        return 0
    lax.fori_loop(0, rows * (D // 16), body, 0)


def _fill_1d(ref, n, val):
    val16 = jnp.full((16,), val, jnp.float32)
    def body(i, _):
        ref[pl.ds(i * 16, 16)] = val16
        return 0
    lax.fori_loop(0, n // 16, body, 0)


@functools.partial(
    pl.kernel,
    out_type=jax.ShapeDtypeStruct((NC * NPAD,), jnp.float32),
    mesh=_mesh,
    scratch_types=[
        pltpu.VMEM((K,), jnp.int32),
        pltpu.VMEM((K,), jnp.float32),
        pltpu.VMEM((((RPT + 15) // 16) * 16,), jnp.float32),
        pltpu.VMEM_SHARED((NPAD,), jnp.float32),
        pltpu.SemaphoreType.DMA,
    ],
)
def _sc_degree(dst_hbm, out_hbm, dst_v, ones_v, zrow_v, acc, sem):
    c = lax.axis_index("c")
    s = lax.axis_index("s")
    wid = s * NC + c

    _fill_1d(ones_v, K, 1.0)
    _fill_1d(zrow_v, ((RPT + 15) // 16) * 16, 0.0)
    pltpu.sync_copy(zrow_v.at[pl.ds(0, RPT)], acc.at[pl.ds(s * RPT, RPT)])
    plsc.subcore_barrier()

    base = wid * CHUNKS * K

    def body(j, _):
        pltpu.sync_copy(dst_hbm.at[pl.ds(base + j * K, K)], dst_v)
        pltpu.sync_copy(ones_v, acc.at[dst_v], add=True)
        return 0

    lax.fori_loop(0, CHUNKS, body, 0)
    plsc.subcore_barrier()
    pltpu.sync_copy(acc.at[pl.ds(s * RPT, RPT)], zrow_v.at[pl.ds(0, RPT)])
    pltpu.sync_copy(zrow_v.at[pl.ds(0, RPT)],
                    out_hbm.at[pl.ds(c * NPAD + s * RPT, RPT)])


@functools.partial(
    pl.kernel,
    out_type=jax.ShapeDtypeStruct((NC, NPAD, D), jnp.float32),
    mesh=_mesh,
    scratch_types=[
        pltpu.VMEM((K,), jnp.int32),
        pltpu.VMEM((K,), jnp.int32),
        pltpu.VMEM((K, D), jnp.float32),
        pltpu.VMEM_SHARED((NPAD, D), jnp.float32),
        pltpu.SemaphoreType.DMA,
    ],
)
def _sc_segsum(y_hbm, src_hbm, dst_hbm, out_hbm, src_v, dst_v, rows_v, acc, sem):
    c = lax.axis_index("c")
    s = lax.axis_index("s")
    wid = s * NC + c

    _fill_2d(rows_v, K, 0.0)
    rbase = s * RPT
    for j in range(RPT // K):
        pltpu.sync_copy(rows_v, acc.at[pl.ds(rbase + j * K, K)])
    rem = RPT % K
    if rem:
        pltpu.sync_copy(rows_v.at[pl.ds(0, rem)],
                        acc.at[pl.ds(rbase + (RPT // K) * K, rem)])
    plsc.subcore_barrier()

    base = wid * CHUNKS * K

    def body(j, _):
        off = base + j * K
        pltpu.sync_copy(src_hbm.at[pl.ds(off, K)], src_v)
        pltpu.sync_copy(dst_hbm.at[pl.ds(off, K)], dst_v)
        pltpu.async_copy(y_hbm.at[src_v], rows_v, sem).wait()
        pltpu.sync_copy(rows_v, acc.at[dst_v], add=True)
        return 0

    lax.fori_loop(0, CHUNKS, body, 0)
    plsc.subcore_barrier()
    pltpu.sync_copy(acc.at[pl.ds(rbase, RPT)], out_hbm.at[c, pl.ds(rbase, RPT)])


_R = 1000


def _tc_dense1_body(x_ref, w_ref, c0_ref, c1_ref, y_ref, dinv_ref):
    deg = c0_ref[...] + c1_ref[...] + 1.0
    dinv = lax.rsqrt(deg)
    dinv_ref[...] = dinv
    y_ref[...] = jnp.dot(x_ref[...], w_ref[...],
                         preferred_element_type=jnp.float32) * dinv


def _tc_dense_mid_body(z0_ref, z1_ref, y_ref, dinv_ref, b_ref, w_ref, out_ref):
    dinv = dinv_ref[...]
    h = dinv * (z0_ref[...] + z1_ref[...] + y_ref[...]) + b_ref[...]
    h = jnp.maximum(h, 0.0)
    out_ref[...] = jnp.dot(h, w_ref[...],
                           preferred_element_type=jnp.float32) * dinv


def _tc_dense_out_body(z0_ref, z1_ref, y_ref, dinv_ref, b_ref, m_ref,
                       wr_ref, br_ref, out_ref):
    h = dinv_ref[...] * (z0_ref[...] + z1_ref[...] + y_ref[...]) + b_ref[...]
    h = jnp.maximum(h, 0.0) * m_ref[...]
    out_ref[...] = jnp.dot(h, wr_ref[...],
                           preferred_element_type=jnp.float32) + br_ref[...]


def _rows_spec(width):
    return pl.BlockSpec((_R, width), lambda i: (i, 0))


def _full_spec(shape):
    return pl.BlockSpec(shape, lambda i: (0,) * len(shape))


def kernel(x, edge_index, target_mask, W1, b1, W2, b2, Wr, br):
    src = edge_index[0]
    dst = edge_index[1]
    pad = EPAD - E
    src_p = jnp.concatenate([src, jnp.zeros((pad,), jnp.int32)])
    dst_p = jnp.concatenate([dst, jnp.full((pad,), N, jnp.int32)])

    cnt = _sc_degree(dst_p)
    c0 = cnt[:N, None]
    c1 = cnt[NPAD:NPAD + N, None]

    grid = N // _R
    y1, dinv = pl.pallas_call(
        _tc_dense1_body,
        grid=(grid,),
        in_specs=[_rows_spec(D), _full_spec((D, D)), _rows_spec(1), _rows_spec(1)],
        out_specs=[_rows_spec(D), _rows_spec(1)],
        out_shape=[jax.ShapeDtypeStruct((N, D), jnp.float32),
                   jax.ShapeDtypeStruct((N, 1), jnp.float32)],
    )(x, W1, c0, c1)

    z = _sc_segsum(y1, src_p, dst_p)

    y2 = pl.pallas_call(
        _tc_dense_mid_body,
        grid=(grid,),
        in_specs=[_rows_spec(D), _rows_spec(D), _rows_spec(D), _rows_spec(1),
                  _full_spec((1, D)), _full_spec((D, D))],
        out_specs=_rows_spec(D),
        out_shape=jax.ShapeDtypeStruct((N, D), jnp.float32),
    )(z[0, :N], z[1, :N], y1, dinv, b1[None, :], W2)

    z2 = _sc_segsum(y2, src_p, dst_p)

    mask_f = target_mask.astype(jnp.float32)[:, None]
    pred = pl.pallas_call(
        _tc_dense_out_body,
        grid=(grid,),
        in_specs=[_rows_spec(D), _rows_spec(D), _rows_spec(D), _rows_spec(1),
                  _full_spec((1, D)), _rows_spec(1),
                  _full_spec((D, DOUT)), _full_spec((1, DOUT))],
        out_specs=_rows_spec(DOUT),
        out_shape=jax.ShapeDtypeStruct((N, DOUT), jnp.float32),
    )(z2[0, :N], z2[1, :N], y2, dinv, b2[None, :], mask_f, Wr, br[None, :])

    return pred
